# Initial kernel scaffold; baseline (speedup 1.0000x reference)
import jax
import jax.numpy as jnp
from jax.experimental import pallas as pl


def kernel(x, edge_index, batch, center_pos, W1l, b1, W1r, W2l, b2, W2r, Wlin, blin):
    raise NotImplementedError("write your pallas kernel here")


# speedup vs baseline: 4.3080x; 4.3080x over previous
import functools

import jax
import jax.numpy as jnp
from jax import lax
from jax.experimental import pallas as pl
from jax.experimental.pallas import tpu as pltpu
from jax.experimental.pallas import tpu_sc as plsc

N = 10000
E = 320000
D = 128
H = 64
G = 64

NC, NS, L = 2, 16, 16
NW = NC * NS
EW = E // NW
CHUNK = 80
NCHUNK = EW // CHUNK
NP = 10240
RW = NP // NS
DROWS = 640


def _tc_matmul2(x, w, m, k, h2):
    nrows = x.shape[0]
    nb = nrows // m

    def body(x_ref, w_ref, a_ref, b_ref):
        xw = jnp.dot(x_ref[...], w_ref[...], preferred_element_type=jnp.float32)
        a_ref[...] = xw[:, :h2]
        b_ref[...] = xw[:, h2:]

    return pl.pallas_call(
        body,
        grid=(nb,),
        in_specs=[
            pl.BlockSpec((m, k), lambda i: (i, 0)),
            pl.BlockSpec((k, 2 * h2), lambda i: (0, 0)),
        ],
        out_specs=[
            pl.BlockSpec((m, h2), lambda i: (i, 0)),
            pl.BlockSpec((m, h2), lambda i: (i, 0)),
        ],
        out_shape=[
            jax.ShapeDtypeStruct((nrows, h2), jnp.float32),
            jax.ShapeDtypeStruct((nrows, h2), jnp.float32),
        ],
    )(x, w)


def _make_sc_edge_agg(want_deg):
    mesh = plsc.VectorSubcoreMesh(core_axis_name="c", subcore_axis_name="s")
    out_type = [jax.ShapeDtypeStruct((NC, NP, H), jnp.float32)]
    if want_deg:
        out_type.append(jax.ShapeDtypeStruct((NW, NP), jnp.float32))
    scratch = [
        pltpu.VMEM((CHUNK,), jnp.int32),
        pltpu.VMEM((CHUNK,), jnp.int32),
        pltpu.VMEM((CHUNK, H), jnp.float32),
        pltpu.VMEM((RW, H), jnp.float32),
        pltpu.VMEM_SHARED((NP, H), jnp.float32),
        pltpu.SemaphoreType.DMA,
    ]
    if want_deg:
        scratch.insert(4, pltpu.VMEM((NP,), jnp.float32))

    def body(rows_hbm, src_hbm, dst_hbm, *refs):
        if want_deg:
            (agg_out, deg_out, srcb, dstb, rowsb, zbuf, degl,
             agg_sh, sem) = refs
        else:
            agg_out, srcb, dstb, rowsb, zbuf, agg_sh, sem = refs
        c = lax.axis_index("c")
        s = lax.axis_index("s")
        wid = s * NC + c

        z16 = jnp.zeros((L,), jnp.float32)

        def zrow(i, carry):
            for q in range(H // L):
                zbuf[i, pl.ds(q * L, L)] = z16
            return carry

        lax.fori_loop(0, RW, zrow, 0)
        if want_deg:
            def zdeg(i, carry):
                degl[pl.ds(i * L, L)] = z16
                return carry

            lax.fori_loop(0, NP // L, zdeg, 0)

        pltpu.sync_copy(zbuf, agg_sh.at[pl.ds(s * RW, RW)])
        plsc.subcore_barrier()

        ones16 = jnp.full((L,), 1.0, jnp.float32)
        ebase = wid * EW

        def chunk(ci, carry):
            base = ebase + ci * CHUNK
            pltpu.sync_copy(src_hbm.at[pl.ds(base, CHUNK)], srcb)
            pltpu.sync_copy(dst_hbm.at[pl.ds(base, CHUNK)], dstb)
            pltpu.async_copy(rows_hbm.at[srcb], rowsb, sem).wait()
            pltpu.sync_copy(rowsb, agg_sh.at[dstb], add=True)
            if want_deg:
                for q in range(CHUNK // L):
                    d = dstb[pl.ds(q * L, L)]
                    plsc.addupdate_scatter(degl, [d], ones16)
            return carry

        lax.fori_loop(0, NCHUNK, chunk, 0)

        plsc.subcore_barrier()
        pltpu.sync_copy(agg_sh.at[pl.ds(s * RW, RW)],
                        agg_out.at[c, pl.ds(s * RW, RW)])
        if want_deg:
            pltpu.sync_copy(degl, deg_out.at[wid])

    return pl.kernel(
        body, out_type=out_type, mesh=mesh, scratch_types=scratch,
        compiler_params=pltpu.CompilerParams(needs_layout_passes=False,
                                             use_tc_tiling_on_sc=False))


_sc_agg_deg = _make_sc_edge_agg(True)
_sc_agg = _make_sc_edge_agg(False)


def _tc_layer(agg_p, deg_p, self_p, b, w, h2):
    m = 1000
    nb = N // m

    def body(a_ref, d_ref, s_ref, b_ref, w_ref, o1_ref, o2_ref):
        agg = a_ref[0] + a_ref[1]
        deg = jnp.sum(d_ref[...], axis=0)
        r = 1.0 / jnp.maximum(deg, 1.0)
        h1 = jnp.maximum(agg * r + b_ref[...] + s_ref[...], 0.0)
        hw = jnp.dot(h1, w_ref[...], preferred_element_type=jnp.float32)
        o1_ref[...] = hw[:, :h2]
        o2_ref[...] = hw[:, h2:]

    return pl.pallas_call(
        body,
        grid=(nb,),
        in_specs=[
            pl.BlockSpec((NC, m, H), lambda i: (0, i, 0)),
            pl.BlockSpec((NW, m, 1), lambda i: (0, i, 0)),
            pl.BlockSpec((m, H), lambda i: (i, 0)),
            pl.BlockSpec((1, H), lambda i: (0, 0)),
            pl.BlockSpec((H, 2 * h2), lambda i: (0, 0)),
        ],
        out_specs=[
            pl.BlockSpec((m, h2), lambda i: (i, 0)),
            pl.BlockSpec((m, h2), lambda i: (i, 0)),
        ],
        out_shape=[
            jax.ShapeDtypeStruct((N, h2), jnp.float32),
            jax.ShapeDtypeStruct((N, h2), jnp.float32),
        ],
    )(agg_p, deg_p, self_p, b, w)


def _tc_final(agg_p, deg_p, self_p, b, wlin, blin):
    m = 1000
    nb = N // m

    def body(a_ref, d_ref, s_ref, b_ref, w_ref, bl_ref, z_ref):
        agg = a_ref[0] + a_ref[1]
        deg = jnp.sum(d_ref[...], axis=0)
        r = 1.0 / jnp.maximum(deg, 1.0)
        h2v = jnp.maximum(agg * r + b_ref[...] + s_ref[...], 0.0)
        z_ref[...] = (jnp.dot(h2v, w_ref[...], preferred_element_type=jnp.float32)
                      + bl_ref[...])

    return pl.pallas_call(
        body,
        grid=(nb,),
        in_specs=[
            pl.BlockSpec((NC, m, H), lambda i: (0, i, 0)),
            pl.BlockSpec((NW, m, 1), lambda i: (0, i, 0)),
            pl.BlockSpec((m, H), lambda i: (i, 0)),
            pl.BlockSpec((1, H), lambda i: (0, 0)),
            pl.BlockSpec((H, 1), lambda i: (0, 0)),
            pl.BlockSpec((1, 1), lambda i: (0, 0)),
        ],
        out_specs=pl.BlockSpec((m, 1), lambda i: (i, 0)),
        out_shape=jax.ShapeDtypeStruct((N, 1), jnp.float32),
    )(agg_p, deg_p, self_p, b, wlin, blin)


def _sc_readout():
    mesh = plsc.VectorSubcoreMesh(core_axis_name="c", subcore_axis_name="s")
    scratch = [
        pltpu.VMEM((N,), jnp.int32),
        pltpu.VMEM((N,), jnp.float32),
        pltpu.VMEM((G,), jnp.int32),
        pltpu.VMEM((G,), jnp.int32),
        pltpu.VMEM((G,), jnp.float32),
    ]

    def body(batch_hbm, cp_hbm, z_hbm, out_hbm, batchl, zl, counts, cpl, outl):
        c = lax.axis_index("c")
        s = lax.axis_index("s")

        @pl.when((c == 0) & (s == 0))
        def _():
            z16i = jnp.zeros((L,), jnp.int32)
            for k in range(G // L):
                counts[pl.ds(k * L, L)] = z16i
            pltpu.sync_copy(batch_hbm, batchl)
            pltpu.sync_copy(cp_hbm, cpl)
            pltpu.sync_copy(z_hbm, zl)
            ones16 = jnp.full((L,), 1, jnp.int32)

            def cnt(i, carry):
                b = batchl[pl.ds(i * L, L)]
                plsc.addupdate_scatter(counts, [b], ones16)
                return carry

            lax.fori_loop(0, N // L, cnt, 0)

            carry = jnp.int32(0)
            for k in range(G // L):
                v = counts[pl.ds(k * L, L)]
                incl = plsc.cumsum(v)
                offs = incl - v + carry
                carry = carry + jnp.sum(v)
                cg = offs + cpl[pl.ds(k * L, L)]
                outl[pl.ds(k * L, L)] = plsc.load_gather(zl, [cg])
            pltpu.sync_copy(outl, out_hbm)

    return pl.kernel(
        body,
        out_type=jax.ShapeDtypeStruct((G,), jnp.float32),
        mesh=mesh,
        scratch_types=scratch,
        compiler_params=pltpu.CompilerParams(needs_layout_passes=False,
                                             use_tc_tiling_on_sc=False),
    )


_sc_read = _sc_readout()


def kernel(x, edge_index, batch, center_pos, W1l, b1, W1r, W2l, b2, W2r,
           Wlin, blin):
    src = edge_index[0]
    dst = edge_index[1]

    w1 = jnp.concatenate([W1l, W1r], axis=1)
    xl, xr = _tc_matmul2(x, w1, 1000, D, H)

    agg1_p, deg_p = _sc_agg_deg(xl, src, dst)
    deg_p3 = deg_p.reshape(NW, NP, 1)

    w2 = jnp.concatenate([W2l, W2r], axis=1)
    hl, hr = _tc_layer(agg1_p, deg_p3, xr, b1.reshape(1, H), w2, H)

    agg2_p, = _sc_agg(hl, src, dst)

    z = _tc_final(agg2_p, deg_p3, hr, b2.reshape(1, H),
                  Wlin, blin.reshape(1, 1))

    out = _sc_read(batch, center_pos, z.reshape(N))
    return out
